# baseline (speedup 1.0000x reference)
import functools

import jax
import jax.numpy as jnp
from jax import lax
from jax.experimental import pallas as pl
from jax.experimental.pallas import tpu as pltpu
from jax.experimental.pallas import tpu_sc as plsc

_DIN, _D1, _D2, _D3, _D4 = 4, 64, 128, 256, 64
_N, _E, _G, _NEF = 1024, 2048, 32, 3
_NW = 32
_BPW = _E // _NW


_LAYERS = (
    ("conv_a", _DIN, 128, _D1, 128, 64),
    ("conv_b", _D1, 128, _D2, 128, 16),
    ("conv_c", _D2, 128, _D3, 256, 8),
    ("conv_d", _D3, 256, _D4, 64, 8),
)


def _mesh():
    return plsc.VectorSubcoreMesh(core_axis_name="c", subcore_axis_name="s")


def _make_gather(din_tab):
    @functools.partial(
        pl.kernel, mesh=_mesh(),
        out_type=jax.ShapeDtypeStruct((_E, din_tab), jnp.float32),
        scratch_types=[pltpu.VMEM((_BPW,), jnp.int32),
                       pltpu.VMEM((_BPW, din_tab), jnp.float32),
                       pltpu.SemaphoreType.DMA],
    )
    def gather(table_hbm, idx_hbm, out_hbm, idx_v, rows_v, sem):
        wid = lax.axis_index("s") * 2 + lax.axis_index("c")
        base = wid * _BPW
        pltpu.sync_copy(idx_hbm.at[pl.ds(base, _BPW)], idx_v)
        pltpu.async_copy(table_hbm.at[idx_v], rows_v, sem).wait()
        pltpu.sync_copy(rows_v, out_hbm.at[pl.ds(base, _BPW)])

    return gather


def _full(shape):
    return pl.BlockSpec(shape, lambda k: (0, 0))


def _dot(a, b):
    return jnp.dot(a, b, preferred_element_type=jnp.float32)


def _onehot_matmul(onehot, dense):
    hi = dense.astype(jnp.bfloat16).astype(jnp.float32)
    return (jnp.dot(onehot, hi, preferred_element_type=jnp.float32)
            + jnp.dot(onehot, dense - hi, preferred_element_type=jnp.float32))


def _pad_cols(v, width):
    if v.shape[1] == width:
        return v
    return jnp.concatenate(
        [v, jnp.zeros((v.shape[0], width - v.shape[1]), jnp.float32)], axis=1)


def _msg_body(din, din_tab, dout, dout_tab, kc, nk, first, *refs):
    (h_ref, xs_ref, dst, ea1, ea2, w1, b1, w2, b2, w3c, b3r, root, bias,
     out_ref, h2_scr, xs_scr, msg_scr) = refs
    k = pl.program_id(0)

    @pl.when(k == 0)
    def _init():
        ea = ea1[...] + ea2[...]
        h1 = jnp.maximum(_dot(ea, w1[...]) + b1[...], 0.0)
        h2 = jnp.maximum(_dot(h1, w2[...]) + b2[...], 0.0)
        h2b = h2.astype(jnp.bfloat16)
        for i in range(nk):
            h2_scr[i] = h2b[:, i * kc:(i + 1) * kc]
        xs = xs_ref[...][:, :din]
        xs_scr[...] = xs.astype(jnp.bfloat16)
        msg_scr[...] = _dot(xs, b3r[...])

    h2c = h2_scr[k].astype(jnp.float32)
    xs = xs_scr[...]
    u = _dot(xs, w3c[...])
    msg = msg_scr[...]
    for j in range(kc):
        msg = msg + h2c[:, j:j + 1] * u[:, j * dout:(j + 1) * dout]
    msg_scr[...] = msg

    @pl.when(k == nk - 1)
    def _fini():
        row_ids = jax.lax.broadcasted_iota(jnp.int32, (_N, _E), 0)
        sca = (dst[...] == row_ids).astype(jnp.float32)
        agg = _onehot_matmul(sca, msg_scr[...])
        h = h_ref[...][:, :din]
        hn = jnp.maximum(agg + _dot(h, root[...]) + bias[...], 0.0)
        out_ref[...] = _pad_cols(hn, dout_tab)


def _head_body(t_ref, batch_row, l1w, l1b, l2w, l2b, l3w, l3b, ow, ob, out_ref):
    h = t_ref[...]
    h = _dot(h, l1w[...]) + l1b[...]
    h = _dot(h, l2w[...]) + l2b[...]
    h = _dot(h, l3w[...]) + l3b[...]
    g_ids = jax.lax.broadcasted_iota(jnp.int32, (_G, _N), 0)
    starts = jnp.sum((batch_row[...] < g_ids).astype(jnp.int32),
                     axis=1, keepdims=True)
    n_ids = jax.lax.broadcasted_iota(jnp.int32, (_G, _N), 1)
    out = ob[...]
    for t in range(3):
        sel = (n_ids == starts + t).astype(jnp.float32)
        ft = _dot(sel, h)
        out = out + _dot(ft, ow[...][64 * t:64 * (t + 1), :])
    out_ref[...] = out


def kernel(x, edge_index, edge_attr1, edge_attr2, batch, params):
    src = edge_index[0]
    dst_row = edge_index[1].reshape(1, _E)
    batch_row = batch.reshape(1, _N)
    t = jnp.pad(x, ((0, 0), (0, 128 - _DIN)))
    for (name, din, din_tab, dout, dout_tab, kc) in _LAYERS:
        p = params[name]
        m = p["mlp"]
        first = name == "conv_a"
        w3c = m["W3"].reshape(64, din, dout).transpose(1, 0, 2)
        w3c = w3c.reshape(din, 64 * dout).astype(jnp.bfloat16)
        b3r = m["b3"].reshape(din, dout)
        nk = 64 // kc

        xs = _make_gather(din_tab)(t, src)

        t = pl.pallas_call(
            functools.partial(_msg_body, din, din_tab, dout, dout_tab, kc,
                              nk, first),
            grid=(nk,),
            in_specs=[
                _full((_N, din_tab)), _full((_E, din_tab)), _full((1, _E)),
                _full((_E, _NEF)), _full((_E, _NEF)),
                _full((_NEF, 64)), _full((1, 64)), _full((64, 64)),
                _full((1, 64)),
                pl.BlockSpec((din, kc * dout), lambda k: (0, k)),
                _full((din, dout)), _full((din, dout)), _full((1, dout)),
            ],
            out_specs=_full((_N, dout_tab)),
            out_shape=jax.ShapeDtypeStruct((_N, dout_tab), jnp.float32),
            scratch_shapes=[pltpu.VMEM((nk, _E, kc), jnp.bfloat16),
                            pltpu.VMEM((_E, din), jnp.bfloat16),
                            pltpu.VMEM((_E, dout), jnp.float32)],
        )(t, xs, dst_row, edge_attr1, edge_attr2,
          m["W1"], m["b1"].reshape(1, 64), m["W2"], m["b2"].reshape(1, 64),
          w3c, b3r, p["root"], p["bias"].reshape(1, dout))

    return pl.pallas_call(
        _head_body,
        out_shape=jax.ShapeDtypeStruct((_G, 1), jnp.float32),
    )(t, batch_row, params["lin1W"], params["lin1b"].reshape(1, 128),
      params["lin2W"], params["lin2b"].reshape(1, 64),
      params["lin3W"], params["lin3b"].reshape(1, 64),
      params["outW"], params["outb"].reshape(1, 1))


# speedup vs baseline: 1.0128x; 1.0128x over previous
import functools

import jax
import jax.numpy as jnp
from jax import lax
from jax.experimental import pallas as pl
from jax.experimental.pallas import tpu as pltpu
from jax.experimental.pallas import tpu_sc as plsc

_DIN, _D1, _D2, _D3, _D4 = 4, 64, 128, 256, 64
_N, _E, _G, _NEF = 1024, 2048, 32, 3
_NW = 32
_BPW = _E // _NW


_LAYERS = (
    ("conv_a", _DIN, 128, _D1, 128, 64),
    ("conv_b", _D1, 128, _D2, 128, 16),
    ("conv_c", _D2, 128, _D3, 256, 8),
    ("conv_d", _D3, 256, _D4, 64, 8),
)


def _mesh():
    return plsc.VectorSubcoreMesh(core_axis_name="c", subcore_axis_name="s")


def _make_gather(din_tab):
    @functools.partial(
        pl.kernel, mesh=_mesh(),
        out_type=jax.ShapeDtypeStruct((_E, din_tab), jnp.float32),
        scratch_types=[pltpu.VMEM((_BPW,), jnp.int32),
                       pltpu.VMEM((_BPW, din_tab), jnp.float32),
                       pltpu.SemaphoreType.DMA],
    )
    def gather(table_hbm, idx_hbm, out_hbm, idx_v, rows_v, sem):
        wid = lax.axis_index("s") * 2 + lax.axis_index("c")
        base = wid * _BPW
        pltpu.sync_copy(idx_hbm.at[pl.ds(base, _BPW)], idx_v)
        pltpu.async_copy(table_hbm.at[idx_v], rows_v, sem).wait()
        pltpu.sync_copy(rows_v, out_hbm.at[pl.ds(base, _BPW)])

    return gather


def _full(shape):
    return pl.BlockSpec(shape, lambda k: (0, 0))


def _dot(a, b):
    return jnp.dot(a, b, preferred_element_type=jnp.float32)


def _onehot_matmul(onehot, dense):
    hi = dense.astype(jnp.bfloat16).astype(jnp.float32)
    return (jnp.dot(onehot, hi, preferred_element_type=jnp.float32)
            + jnp.dot(onehot, dense - hi, preferred_element_type=jnp.float32))


def _pad_cols(v, width):
    if v.shape[1] == width:
        return v
    return jnp.concatenate(
        [v, jnp.zeros((v.shape[0], width - v.shape[1]), jnp.float32)], axis=1)


def _msg_body(din, din_tab, dout, dout_tab, kc, nk, head, *refs):
    if head:
        (h_ref, xs_ref, dst, ea1, ea2, w1, b1, w2, b2, w3c, b3r, root, bias,
         batch_row, l1w, l1b, l2w, l2b, l3w, l3b, ow, ob,
         out_ref, h2_scr, xs_scr, msg_scr) = refs
    else:
        (h_ref, xs_ref, dst, ea1, ea2, w1, b1, w2, b2, w3c, b3r, root, bias,
         out_ref, h2_scr, xs_scr, msg_scr) = refs
    k = pl.program_id(0)

    @pl.when(k == 0)
    def _init():
        ea = ea1[...] + ea2[...]
        h1 = jnp.maximum(_dot(ea, w1[...]) + b1[...], 0.0)
        h2 = jnp.maximum(_dot(h1, w2[...]) + b2[...], 0.0)
        h2b = h2.astype(jnp.bfloat16)
        for i in range(nk):
            h2_scr[i] = h2b[:, i * kc:(i + 1) * kc]
        xs = xs_ref[...][:, :din]
        xs_scr[...] = xs.astype(jnp.bfloat16)
        msg_scr[...] = _dot(xs, b3r[...])

    h2c = h2_scr[k].astype(jnp.float32)
    xs = xs_scr[...]
    u = _dot(xs, w3c[...])
    msg = msg_scr[...]
    for j in range(kc):
        msg = msg + h2c[:, j:j + 1] * u[:, j * dout:(j + 1) * dout]
    msg_scr[...] = msg

    @pl.when(k == nk - 1)
    def _fini():
        row_ids = jax.lax.broadcasted_iota(jnp.int32, (_N, _E), 0)
        sca = (dst[...] == row_ids).astype(jnp.float32)
        agg = _onehot_matmul(sca, msg_scr[...])
        h = h_ref[...][:, :din]
        hn = jnp.maximum(agg + _dot(h, root[...]) + bias[...], 0.0)
        if not head:
            out_ref[...] = _pad_cols(hn, dout_tab)
        else:
            h = _dot(hn, l1w[...]) + l1b[...]
            h = _dot(h, l2w[...]) + l2b[...]
            h = _dot(h, l3w[...]) + l3b[...]
            g_ids = jax.lax.broadcasted_iota(jnp.int32, (_G, _N), 0)
            starts = jnp.sum((batch_row[...] < g_ids).astype(jnp.int32),
                             axis=1, keepdims=True)
            n_ids = jax.lax.broadcasted_iota(jnp.int32, (_G, _N), 1)
            out = ob[...]
            for t in range(3):
                sel = (n_ids == starts + t).astype(jnp.float32)
                ft = _dot(sel, h)
                out = out + _dot(ft, ow[...][64 * t:64 * (t + 1), :])
            out_ref[...] = out


def kernel(x, edge_index, edge_attr1, edge_attr2, batch, params):
    src = edge_index[0]
    dst_row = edge_index[1].reshape(1, _E)
    batch_row = batch.reshape(1, _N)
    t = jnp.pad(x, ((0, 0), (0, 128 - _DIN)))
    for (name, din, din_tab, dout, dout_tab, kc) in _LAYERS:
        p = params[name]
        m = p["mlp"]
        head = name == "conv_d"
        w3c = m["W3"].reshape(64, din, dout).transpose(1, 0, 2)
        w3c = w3c.reshape(din, 64 * dout).astype(jnp.bfloat16)
        b3r = m["b3"].reshape(din, dout)
        nk = 64 // kc

        xs = _make_gather(din_tab)(t, src)

        ops = [t, xs, dst_row, edge_attr1, edge_attr2,
               m["W1"], m["b1"].reshape(1, 64), m["W2"], m["b2"].reshape(1, 64),
               w3c, b3r, p["root"], p["bias"].reshape(1, dout)]
        in_specs = [
            _full((_N, din_tab)), _full((_E, din_tab)), _full((1, _E)),
            _full((_E, _NEF)), _full((_E, _NEF)),
            _full((_NEF, 64)), _full((1, 64)), _full((64, 64)),
            _full((1, 64)),
            pl.BlockSpec((din, kc * dout), lambda k: (0, k)),
            _full((din, dout)), _full((din, dout)), _full((1, dout)),
        ]
        if head:
            ops += [batch_row, params["lin1W"], params["lin1b"].reshape(1, 128),
                    params["lin2W"], params["lin2b"].reshape(1, 64),
                    params["lin3W"], params["lin3b"].reshape(1, 64),
                    params["outW"], params["outb"].reshape(1, 1)]
            in_specs += [_full((1, _N)), _full((_D4, 128)), _full((1, 128)),
                         _full((128, 64)), _full((1, 64)), _full((64, 64)),
                         _full((1, 64)), _full((192, 1)), _full((1, 1))]
            out_shape = jax.ShapeDtypeStruct((_G, 1), jnp.float32)
            out_specs = _full((_G, 1))
        else:
            out_shape = jax.ShapeDtypeStruct((_N, dout_tab), jnp.float32)
            out_specs = _full((_N, dout_tab))
        t = pl.pallas_call(
            functools.partial(_msg_body, din, din_tab, dout, dout_tab, kc,
                              nk, head),
            grid=(nk,),
            in_specs=in_specs,
            out_specs=out_specs,
            out_shape=out_shape,
            scratch_shapes=[pltpu.VMEM((nk, _E, kc), jnp.bfloat16),
                            pltpu.VMEM((_E, din), jnp.bfloat16),
                            pltpu.VMEM((_E, dout), jnp.float32)],
        )(*ops)

    return t
